# Initial kernel scaffold; baseline (speedup 1.0000x reference)
import jax
import jax.numpy as jnp
from jax.experimental import pallas as pl


def kernel(f_atoms, f_bonds, a2b, b2a, b2revb, a_scope, params):
    raise NotImplementedError("write your pallas kernel here")


# speedup vs baseline: 1.1575x; 1.1575x over previous
import functools
import math

import jax
import jax.numpy as jnp
from jax import lax
from jax.experimental import pallas as pl
from jax.experimental.pallas import tpu as pltpu
from jax.experimental.pallas import tpu_sc as plsc

H = 128
AF = 128
BF = 16
NA = 10000
NB = 160000
NEI = 16
NM = 100
MA = 100
DEPTH = 4
NH = 4
DH = H // NH
FF = 2 * H

NC = 2
NS = 16
NW = NC * NS

NAP = 10240
CA = 16
APW = NAP // NW
NCHA = APW // CA

BPW = NB // NW
CB = 40
NCHB = BPW // CB

def _sc_mesh():
    return plsc.VectorSubcoreMesh(core_axis_name="c", subcore_axis_name="s")


def _wid():
    return lax.axis_index("s") * NC + lax.axis_index("c")


def _sc_gather_rows(table, idx):
    n = idx.shape[0]
    per_w = n // NW
    nch = per_w // CB

    @functools.partial(
        pl.kernel,
        mesh=_sc_mesh(),
        out_type=jax.ShapeDtypeStruct((n, H), jnp.float32),
        scratch_types=[
            pltpu.VMEM((CB,), jnp.int32),
            pltpu.VMEM((CB, H), jnp.float32),
            pltpu.SemaphoreType.DMA,
        ],
    )
    def k(table_hbm, idx_hbm, out_hbm, i_v, r_v, sem):
        w = _wid()

        def chunk(ci, carry):
            base = w * per_w + ci * CB
            pltpu.sync_copy(idx_hbm.at[pl.ds(base, CB)], i_v)
            pltpu.async_copy(table_hbm.at[i_v], r_v, sem).wait()
            pltpu.sync_copy(r_v, out_hbm.at[pl.ds(base, CB)])
            return carry

        lax.fori_loop(0, nch, chunk, 0)

    return k(table, idx)


def _sc_gather_sum(msg, a2b_flat):
    @functools.partial(
        pl.kernel,
        mesh=_sc_mesh(),
        out_type=jax.ShapeDtypeStruct((NAP, H), jnp.float32),
        scratch_types=[
            pltpu.VMEM((CA * NEI,), jnp.int32),
            pltpu.VMEM((CA * NEI, H), jnp.float32),
            pltpu.VMEM((CA, H), jnp.float32),
            pltpu.SemaphoreType.DMA,
        ],
    )
    def k(msg_hbm, a2b_hbm, out_hbm, i_v, r_v, acc_v, sem):
        w = _wid()

        def chunk(ci, carry):
            abase = w * APW + ci * CA
            pltpu.sync_copy(a2b_hbm.at[pl.ds(abase * NEI, CA * NEI)], i_v)
            pltpu.async_copy(msg_hbm.at[i_v], r_v, sem).wait()

            def atom(a, c2):
                def nb(j, accs):
                    return tuple(
                        accs[kk] + r_v[a * NEI + j, pl.ds(kk * 16, 16)]
                        for kk in range(8)
                    )

                accs = lax.fori_loop(
                    0, NEI, nb,
                    tuple(jnp.zeros((16,), jnp.float32) for _ in range(8)),
                )
                for kk in range(8):
                    acc_v[a, pl.ds(kk * 16, 16)] = accs[kk]
                return c2

            lax.fori_loop(0, CA, atom, 0)
            pltpu.sync_copy(acc_v, out_hbm.at[pl.ds(abase, CA)])
            return carry

        lax.fori_loop(0, NCHA, chunk, 0)

    return k(msg, a2b_flat)


def _sc_gather_sub(nei, msg, b2a, b2revb):
    @functools.partial(
        pl.kernel,
        mesh=_sc_mesh(),
        out_type=jax.ShapeDtypeStruct((NB, H), jnp.float32),
        scratch_types=[
            pltpu.VMEM((CB,), jnp.int32),
            pltpu.VMEM((CB,), jnp.int32),
            pltpu.VMEM((CB, H), jnp.float32),
            pltpu.VMEM((CB, H), jnp.float32),
            pltpu.SemaphoreType.DMA,
            pltpu.SemaphoreType.DMA,
        ],
    )
    def k(nei_hbm, msg_hbm, b2a_hbm, b2revb_hbm, out_hbm,
          ia_v, ib_v, ra_v, rb_v, sema, semb):
        w = _wid()

        def chunk(ci, carry):
            base = w * BPW + ci * CB
            pltpu.sync_copy(b2a_hbm.at[pl.ds(base, CB)], ia_v)
            pltpu.sync_copy(b2revb_hbm.at[pl.ds(base, CB)], ib_v)
            cpa = pltpu.async_copy(nei_hbm.at[ia_v], ra_v, sema)
            cpb = pltpu.async_copy(msg_hbm.at[ib_v], rb_v, semb)
            cpa.wait()
            cpb.wait()

            def row(r, c2):
                for kk in range(8):
                    sl = pl.ds(kk * 16, 16)
                    ra_v[r, sl] = ra_v[r, sl] - rb_v[r, sl]
                return c2

            lax.fori_loop(0, CB, row, 0)
            pltpu.sync_copy(ra_v, out_hbm.at[pl.ds(base, CB)])
            return carry

        lax.fori_loop(0, NCHB, chunk, 0)

    return k(nei, msg, b2a, b2revb)


def _ln(x, g, b):
    m = jnp.mean(x, -1, keepdims=True)
    v = jnp.mean((x - m) ** 2, -1, keepdims=True)
    return (x - m) * lax.rsqrt(v + 1e-5) * g + b


def _gelu(x):
    return 0.5 * x * (1.0 + lax.erf(x * (1.0 / math.sqrt(2.0))))


BLK = 2000
BLKA = 2000


def _tc_init(ga, f_bonds, wia, wib, bi):
    def body(ga_ref, fb_ref, wa_ref, wb_ref, bi_ref, o_ref):
        x = (jnp.dot(ga_ref[...], wa_ref[...], preferred_element_type=jnp.float32)
             + jnp.dot(fb_ref[...], wb_ref[...], preferred_element_type=jnp.float32)
             + bi_ref[...])
        o_ref[...] = _gelu(x)

    return pl.pallas_call(
        body,
        grid=(NB // BLK,),
        in_specs=[
            pl.BlockSpec((BLK, AF), lambda i: (i, 0)),
            pl.BlockSpec((BLK, BF), lambda i: (i, 0)),
            pl.BlockSpec((AF, H), lambda i: (0, 0)),
            pl.BlockSpec((BF, H), lambda i: (0, 0)),
            pl.BlockSpec((1, H), lambda i: (0, 0)),
        ],
        out_specs=pl.BlockSpec((BLK, H), lambda i: (i, 0)),
        out_shape=jax.ShapeDtypeStruct((NB, H), jnp.float32),
    )(ga, f_bonds, wia, wib, bi)


def _tc_depth(nm, msg, wh, bh, g, b):
    def body(nm_ref, msg_ref, wh_ref, bh_ref, g_ref, b_ref, o_ref):
        xn = _ln(nm_ref[...], g_ref[...], b_ref[...])
        y = jnp.dot(xn, wh_ref[...], preferred_element_type=jnp.float32) + bh_ref[...]
        o_ref[...] = msg_ref[...] + _gelu(y)

    return pl.pallas_call(
        body,
        grid=(NB // BLK,),
        in_specs=[
            pl.BlockSpec((BLK, H), lambda i: (i, 0)),
            pl.BlockSpec((BLK, H), lambda i: (i, 0)),
            pl.BlockSpec((H, H), lambda i: (0, 0)),
            pl.BlockSpec((1, H), lambda i: (0, 0)),
            pl.BlockSpec((1, H), lambda i: (0, 0)),
            pl.BlockSpec((1, H), lambda i: (0, 0)),
        ],
        out_specs=pl.BlockSpec((BLK, H), lambda i: (i, 0)),
        out_shape=jax.ShapeDtypeStruct((NB, H), jnp.float32),
    )(nm, msg, wh, bh, g, b)


def _tc_atom(f_atoms, a_msg, woa, wob, bo, ang, anb):
    def body(fa_ref, am_ref, wa_ref, wb_ref, bo_ref, g_ref, b_ref, o_ref):
        x = (jnp.dot(fa_ref[...], wa_ref[...], preferred_element_type=jnp.float32)
             + jnp.dot(am_ref[...], wb_ref[...], preferred_element_type=jnp.float32)
             + bo_ref[...])
        o_ref[...] = _ln(_gelu(x), g_ref[...], b_ref[...])

    return pl.pallas_call(
        body,
        grid=(NA // BLKA,),
        in_specs=[
            pl.BlockSpec((BLKA, AF), lambda i: (i, 0)),
            pl.BlockSpec((BLKA, H), lambda i: (i, 0)),
            pl.BlockSpec((AF, H), lambda i: (0, 0)),
            pl.BlockSpec((H, H), lambda i: (0, 0)),
            pl.BlockSpec((1, H), lambda i: (0, 0)),
            pl.BlockSpec((1, H), lambda i: (0, 0)),
            pl.BlockSpec((1, H), lambda i: (0, 0)),
        ],
        out_specs=pl.BlockSpec((BLKA, H), lambda i: (i, 0)),
        out_shape=jax.ShapeDtypeStruct((NA, H), jnp.float32),
    )(f_atoms, a_msg, woa, wob, bo, ang, anb)


def _tc_attn(x3, wqt, bq, wkt, bk, wvt, bv, waot, bao,
             ln1g, ln1b, ln2g, ln2b, w1t, b1, w2t, b2, rq, wkrt, bkr):
    def body(x_ref, wq_ref, bq_ref, wk_ref, bk_ref, wv_ref, bv_ref,
             wao_ref, bao_ref, g1_ref, b1n_ref, g2_ref, b2n_ref,
             w1_ref, bf1_ref, w2_ref, bf2_ref, rq_ref, wkr_ref, bkr_ref,
             o_ref):
        x0 = x_ref[0]
        h = _ln(x0, g1_ref[...], b1n_ref[...])
        q = jnp.dot(h, wq_ref[...], preferred_element_type=jnp.float32) + bq_ref[...]
        kk = jnp.dot(h, wk_ref[...], preferred_element_type=jnp.float32) + bk_ref[...]
        v = jnp.dot(h, wv_ref[...], preferred_element_type=jnp.float32) + bv_ref[...]
        scale = 1.0 / math.sqrt(DH)
        parts = []
        for hd in range(NH):
            sl = slice(hd * DH, (hd + 1) * DH)
            qh = q[:, sl]
            khd = kk[:, sl]
            vh = v[:, sl]
            s = lax.dot_general(qh, khd, (((1,), (1,)), ((), ())),
                                preferred_element_type=jnp.float32) * scale
            p = jax.nn.softmax(s, axis=-1)
            parts.append(jnp.dot(p, vh, preferred_element_type=jnp.float32))
        att = jnp.concatenate(parts, axis=1)
        ao = jnp.dot(att, wao_ref[...], preferred_element_type=jnp.float32) + bao_ref[...]
        x = x0 + ao
        h2 = _ln(x, g2_ref[...], b2n_ref[...])
        ffn = jnp.dot(_gelu(jnp.dot(h2, w1_ref[...], preferred_element_type=jnp.float32)
                            + bf1_ref[...]),
                      w2_ref[...], preferred_element_type=jnp.float32)
        x = x + ffn + bf2_ref[...]
        keys = jnp.dot(x, wkr_ref[...], preferred_element_type=jnp.float32) + bkr_ref[...]
        s = lax.dot_general(rq_ref[...], keys, (((1,), (1,)), ((), ())),
                            preferred_element_type=jnp.float32)
        w = jax.nn.softmax(s, axis=-1)
        o_ref[0] = jnp.dot(w, x, preferred_element_type=jnp.float32)

    full = lambda shape: pl.BlockSpec(shape, lambda i: tuple(0 for _ in shape))
    return pl.pallas_call(
        body,
        grid=(NM,),
        in_specs=[
            pl.BlockSpec((1, MA, H), lambda i: (i, 0, 0)),
            full((H, H)), full((1, H)),
            full((H, H)), full((1, H)),
            full((H, H)), full((1, H)),
            full((H, H)), full((1, H)),
            full((1, H)), full((1, H)),
            full((1, H)), full((1, H)),
            full((H, FF)), full((1, FF)),
            full((FF, H)), full((1, H)),
            full((1, H)), full((H, H)), full((1, H)),
        ],
        out_specs=pl.BlockSpec((1, 1, H), lambda i: (i, 0, 0)),
        out_shape=jax.ShapeDtypeStruct((NM, 1, H), jnp.float32),
    )(x3, wqt, bq, wkt, bk, wvt, bv, waot, bao,
      ln1g, ln1b, ln2g, ln2b, w1t, b1, w2t, b2, rq, wkrt, bkr)


def kernel(f_atoms, f_bonds, a2b, b2a, b2revb, a_scope, params):
    p = params
    r2 = lambda a: a.reshape(1, -1)

    wia = p['W_i'][:, :AF].T
    wib = p['W_i'][:, AF:].T
    woa = p['W_o'][:, :AF].T
    wob = p['W_o'][:, AF:].T

    a2b_flat = jnp.concatenate(
        [a2b, jnp.zeros((NAP - NA, NEI), a2b.dtype)], axis=0
    ).reshape(-1)

    ga = _sc_gather_rows(f_atoms, b2a)
    msg = _tc_init(ga, f_bonds, wia, wib, r2(p['b_i']))

    for t in range(DEPTH - 1):
        nei = _sc_gather_sum(msg, a2b_flat)
        nm = _sc_gather_sub(nei, msg, b2a, b2revb)
        msg = _tc_depth(nm, msg, p['W_h'][t].T, r2(p['b_h'][t]),
                        r2(p['msg_g'][t]), r2(p['msg_b'][t]))

    a_msg = _sc_gather_sum(msg, a2b_flat)[:NA]
    ah = _tc_atom(f_atoms, a_msg, woa, wob, r2(p['b_o']),
                  r2(p['an_g']), r2(p['an_b']))

    x3 = ah.reshape(NM, MA, H)
    out = _tc_attn(
        x3, p['Wq'].T, r2(p['bq']), p['Wk'].T, r2(p['bk']),
        p['Wv'].T, r2(p['bv']), p['Wao'].T, r2(p['bao']),
        r2(p['ln1_g']), r2(p['ln1_b']), r2(p['ln2_g']), r2(p['ln2_b']),
        p['W1'].T, r2(p['b1']), p['W2'].T, r2(p['b2']),
        p['rq'].reshape(1, H), p['Wkr'].T, r2(p['bkr']))
    return out.reshape(NM, H)


# speedup vs baseline: 1.6104x; 1.3913x over previous
import functools
import math

import jax
import jax.numpy as jnp
from jax import lax
from jax.experimental import pallas as pl
from jax.experimental.pallas import tpu as pltpu
from jax.experimental.pallas import tpu_sc as plsc

H = 128
AF = 128
BF = 16
NA = 10000
NB = 160000
NEI = 16
NM = 100
MA = 100
DEPTH = 4
NH = 4
DH = H // NH
FF = 2 * H

NC = 2
NS = 16
NW = NC * NS

NAP = 10240
CA = 8
APW = NAP // NW
NCHA = APW // CA

BPW = NB // NW
CB = 40
NCHB = BPW // CB

def _sc_mesh():
    return plsc.VectorSubcoreMesh(core_axis_name="c", subcore_axis_name="s")


def _wid():
    return lax.axis_index("s") * NC + lax.axis_index("c")


def _sc_gather_rows(table, idx):
    n = idx.shape[0]
    per_w = n // NW
    nch = per_w // CB
    npairs = (nch + 1) // 2

    @functools.partial(
        pl.kernel,
        mesh=_sc_mesh(),
        out_type=jax.ShapeDtypeStruct((n, H), jnp.float32),
        scratch_types=[
            pltpu.VMEM((per_w,), jnp.int32),
            pltpu.VMEM((CB, H), jnp.float32),
            pltpu.VMEM((CB, H), jnp.float32),
            pltpu.SemaphoreType.DMA,
            pltpu.SemaphoreType.DMA,
            pltpu.SemaphoreType.DMA,
            pltpu.SemaphoreType.DMA,
        ],
    )
    def k(table_hbm, idx_hbm, out_hbm, i_v, r0, r1, sg0, sg1, ss0, ss1):
        w = _wid()
        base_w = w * per_w
        pltpu.sync_copy(idx_hbm.at[pl.ds(base_w, per_w)], i_v)
        bufs = ((r0, sg0, ss0), (r1, sg1, ss1))

        def issue(ci, b):
            r, sg, _ = bufs[b]
            pltpu.async_copy(table_hbm.at[i_v.at[pl.ds(ci * CB, CB)]], r, sg)

        issue(0, 0)
        issue(1, 1)

        def pair(pi, carry):
            for b in range(2):
                r, sg, ss = bufs[b]
                ci = pi * 2 + b

                @pl.when(ci < nch)
                def _():
                    pltpu.make_async_copy(table_hbm.at[pl.ds(0, CB)], r, sg).wait()
                    pltpu.async_copy(r, out_hbm.at[pl.ds(base_w + ci * CB, CB)], ss)
                    pltpu.make_async_copy(r, out_hbm.at[pl.ds(0, CB)], ss).wait()

                    @pl.when(ci + 2 < nch)
                    def _():
                        issue(ci + 2, b)
            return carry

        lax.fori_loop(0, npairs, pair, 0)

    return k(table, idx)


def _sc_gather_sum(msg, a2b_flat):
    @functools.partial(
        pl.kernel,
        mesh=_sc_mesh(),
        out_type=jax.ShapeDtypeStruct((NAP, H), jnp.float32),
        scratch_types=[
            pltpu.VMEM((APW * NEI,), jnp.int32),
            pltpu.VMEM((CA * NEI, H), jnp.float32),
            pltpu.VMEM((CA * NEI, H), jnp.float32),
            pltpu.VMEM((CA, H), jnp.float32),
            pltpu.VMEM((CA, H), jnp.float32),
            pltpu.SemaphoreType.DMA,
            pltpu.SemaphoreType.DMA,
            pltpu.SemaphoreType.DMA,
            pltpu.SemaphoreType.DMA,
        ],
    )
    def k(msg_hbm, a2b_hbm, out_hbm, i_v, r0, r1, o0, o1, sg0, sg1, ss0, ss1):
        w = _wid()
        abase_w = w * APW
        pltpu.sync_copy(a2b_hbm.at[pl.ds(abase_w * NEI, APW * NEI)], i_v)
        bufs = ((r0, o0, sg0, ss0), (r1, o1, sg1, ss1))

        def issue(ci, b):
            r, _, sg, _ = bufs[b]
            pltpu.async_copy(
                msg_hbm.at[i_v.at[pl.ds(ci * CA * NEI, CA * NEI)]], r, sg)

        issue(0, 0)
        issue(1, 1)
        npairs = (NCHA + 1) // 2

        def pair(pi, carry):
            for b in range(2):
                r, o, sg, ss = bufs[b]
                ci = pi * 2 + b

                @pl.when(ci < NCHA)
                def _():
                    pltpu.make_async_copy(
                        msg_hbm.at[pl.ds(0, CA * NEI)], r, sg).wait()

                    @pl.when(ci >= 2)
                    def _():
                        pltpu.make_async_copy(
                            o, out_hbm.at[pl.ds(0, CA)], ss).wait()

                    def atom(a, c2):
                        def nb(j, accs):
                            return tuple(
                                accs[kk] + r[a * NEI + j, pl.ds(kk * 16, 16)]
                                for kk in range(8)
                            )

                        accs = lax.fori_loop(
                            0, NEI, nb,
                            tuple(jnp.zeros((16,), jnp.float32)
                                  for _ in range(8)),
                        )
                        for kk in range(8):
                            o[a, pl.ds(kk * 16, 16)] = accs[kk]
                        return c2

                    lax.fori_loop(0, CA, atom, 0)
                    pltpu.async_copy(
                        o, out_hbm.at[pl.ds(abase_w + ci * CA, CA)], ss)

                    @pl.when(ci + 2 < NCHA)
                    def _():
                        issue(ci + 2, b)
            return carry

        lax.fori_loop(0, npairs, pair, 0)
        for b in range(2):
            _, o, _, ss = bufs[b]
            pltpu.make_async_copy(o, out_hbm.at[pl.ds(0, CA)], ss).wait()

    return k(msg, a2b_flat)


def _sc_gather_sub(nei, msg, b2a, b2revb):
    @functools.partial(
        pl.kernel,
        mesh=_sc_mesh(),
        out_type=jax.ShapeDtypeStruct((NB, H), jnp.float32),
        scratch_types=[
            pltpu.VMEM((BPW,), jnp.int32),
            pltpu.VMEM((BPW,), jnp.int32),
            pltpu.VMEM((CB, H), jnp.float32),
            pltpu.VMEM((CB, H), jnp.float32),
            pltpu.VMEM((CB, H), jnp.float32),
            pltpu.VMEM((CB, H), jnp.float32),
            pltpu.VMEM((CB, H), jnp.float32),
            pltpu.VMEM((CB, H), jnp.float32),
            pltpu.SemaphoreType.DMA,
            pltpu.SemaphoreType.DMA,
            pltpu.SemaphoreType.DMA,
            pltpu.SemaphoreType.DMA,
            pltpu.SemaphoreType.DMA,
            pltpu.SemaphoreType.DMA,
        ],
    )
    def k(nei_hbm, msg_hbm, b2a_hbm, b2revb_hbm, out_hbm,
          ia_v, ib_v, ra0, ra1, rb0, rb1, o0, o1,
          sa0, sa1, sb0, sb1, ss0, ss1):
        w = _wid()
        base_w = w * BPW
        pltpu.sync_copy(b2a_hbm.at[pl.ds(base_w, BPW)], ia_v)
        pltpu.sync_copy(b2revb_hbm.at[pl.ds(base_w, BPW)], ib_v)
        bufs = ((ra0, rb0, o0, sa0, sb0, ss0), (ra1, rb1, o1, sa1, sb1, ss1))

        def issue(ci, b):
            ra, rb, _, sa, sb, _ = bufs[b]
            pltpu.async_copy(nei_hbm.at[ia_v.at[pl.ds(ci * CB, CB)]], ra, sa)
            pltpu.async_copy(msg_hbm.at[ib_v.at[pl.ds(ci * CB, CB)]], rb, sb)

        issue(0, 0)
        issue(1, 1)
        npairs = (NCHB + 1) // 2

        def pair(pi, carry):
            for b in range(2):
                ra, rb, o, sa, sb, ss = bufs[b]
                ci = pi * 2 + b

                @pl.when(ci < NCHB)
                def _():
                    pltpu.make_async_copy(
                        nei_hbm.at[pl.ds(0, CB)], ra, sa).wait()
                    pltpu.make_async_copy(
                        msg_hbm.at[pl.ds(0, CB)], rb, sb).wait()

                    @pl.when(ci >= 2)
                    def _():
                        pltpu.make_async_copy(
                            o, out_hbm.at[pl.ds(0, CB)], ss).wait()

                    def row(r_i, c2):
                        for kk in range(8):
                            sl = pl.ds(kk * 16, 16)
                            o[r_i, sl] = ra[r_i, sl] - rb[r_i, sl]
                        return c2

                    lax.fori_loop(0, CB, row, 0)
                    pltpu.async_copy(
                        o, out_hbm.at[pl.ds(base_w + ci * CB, CB)], ss)

                    @pl.when(ci + 2 < NCHB)
                    def _():
                        issue(ci + 2, b)
            return carry

        lax.fori_loop(0, npairs, pair, 0)
        for b in range(2):
            _, _, o, _, _, ss = bufs[b]
            pltpu.make_async_copy(o, out_hbm.at[pl.ds(0, CB)], ss).wait()

    return k(nei, msg, b2a, b2revb)


def _ln(x, g, b):
    m = jnp.mean(x, -1, keepdims=True)
    v = jnp.mean((x - m) ** 2, -1, keepdims=True)
    return (x - m) * lax.rsqrt(v + 1e-5) * g + b


def _gelu(x):
    return 0.5 * x * (1.0 + lax.erf(x * (1.0 / math.sqrt(2.0))))


BLK = 2000
BLKA = 2000


def _tc_init(ga, f_bonds, wia, wib, bi):
    def body(ga_ref, fb_ref, wa_ref, wb_ref, bi_ref, o_ref):
        x = (jnp.dot(ga_ref[...], wa_ref[...], preferred_element_type=jnp.float32)
             + jnp.dot(fb_ref[...], wb_ref[...], preferred_element_type=jnp.float32)
             + bi_ref[...])
        o_ref[...] = _gelu(x)

    return pl.pallas_call(
        body,
        grid=(NB // BLK,),
        in_specs=[
            pl.BlockSpec((BLK, AF), lambda i: (i, 0)),
            pl.BlockSpec((BLK, BF), lambda i: (i, 0)),
            pl.BlockSpec((AF, H), lambda i: (0, 0)),
            pl.BlockSpec((BF, H), lambda i: (0, 0)),
            pl.BlockSpec((1, H), lambda i: (0, 0)),
        ],
        out_specs=pl.BlockSpec((BLK, H), lambda i: (i, 0)),
        out_shape=jax.ShapeDtypeStruct((NB, H), jnp.float32),
    )(ga, f_bonds, wia, wib, bi)


def _tc_depth(nm, msg, wh, bh, g, b):
    def body(nm_ref, msg_ref, wh_ref, bh_ref, g_ref, b_ref, o_ref):
        xn = _ln(nm_ref[...], g_ref[...], b_ref[...])
        y = jnp.dot(xn, wh_ref[...], preferred_element_type=jnp.float32) + bh_ref[...]
        o_ref[...] = msg_ref[...] + _gelu(y)

    return pl.pallas_call(
        body,
        grid=(NB // BLK,),
        in_specs=[
            pl.BlockSpec((BLK, H), lambda i: (i, 0)),
            pl.BlockSpec((BLK, H), lambda i: (i, 0)),
            pl.BlockSpec((H, H), lambda i: (0, 0)),
            pl.BlockSpec((1, H), lambda i: (0, 0)),
            pl.BlockSpec((1, H), lambda i: (0, 0)),
            pl.BlockSpec((1, H), lambda i: (0, 0)),
        ],
        out_specs=pl.BlockSpec((BLK, H), lambda i: (i, 0)),
        out_shape=jax.ShapeDtypeStruct((NB, H), jnp.float32),
    )(nm, msg, wh, bh, g, b)


def _tc_atom(f_atoms, a_msg, woa, wob, bo, ang, anb):
    def body(fa_ref, am_ref, wa_ref, wb_ref, bo_ref, g_ref, b_ref, o_ref):
        x = (jnp.dot(fa_ref[...], wa_ref[...], preferred_element_type=jnp.float32)
             + jnp.dot(am_ref[...], wb_ref[...], preferred_element_type=jnp.float32)
             + bo_ref[...])
        o_ref[...] = _ln(_gelu(x), g_ref[...], b_ref[...])

    return pl.pallas_call(
        body,
        grid=(NA // BLKA,),
        in_specs=[
            pl.BlockSpec((BLKA, AF), lambda i: (i, 0)),
            pl.BlockSpec((BLKA, H), lambda i: (i, 0)),
            pl.BlockSpec((AF, H), lambda i: (0, 0)),
            pl.BlockSpec((H, H), lambda i: (0, 0)),
            pl.BlockSpec((1, H), lambda i: (0, 0)),
            pl.BlockSpec((1, H), lambda i: (0, 0)),
            pl.BlockSpec((1, H), lambda i: (0, 0)),
        ],
        out_specs=pl.BlockSpec((BLKA, H), lambda i: (i, 0)),
        out_shape=jax.ShapeDtypeStruct((NA, H), jnp.float32),
    )(f_atoms, a_msg, woa, wob, bo, ang, anb)


def _tc_attn(x3, wqt, bq, wkt, bk, wvt, bv, waot, bao,
             ln1g, ln1b, ln2g, ln2b, w1t, b1, w2t, b2, rq, wkrt, bkr):
    def body(x_ref, wq_ref, bq_ref, wk_ref, bk_ref, wv_ref, bv_ref,
             wao_ref, bao_ref, g1_ref, b1n_ref, g2_ref, b2n_ref,
             w1_ref, bf1_ref, w2_ref, bf2_ref, rq_ref, wkr_ref, bkr_ref,
             o_ref):
        x0 = x_ref[0]
        h = _ln(x0, g1_ref[...], b1n_ref[...])
        q = jnp.dot(h, wq_ref[...], preferred_element_type=jnp.float32) + bq_ref[...]
        kk = jnp.dot(h, wk_ref[...], preferred_element_type=jnp.float32) + bk_ref[...]
        v = jnp.dot(h, wv_ref[...], preferred_element_type=jnp.float32) + bv_ref[...]
        scale = 1.0 / math.sqrt(DH)
        parts = []
        for hd in range(NH):
            sl = slice(hd * DH, (hd + 1) * DH)
            qh = q[:, sl]
            khd = kk[:, sl]
            vh = v[:, sl]
            s = lax.dot_general(qh, khd, (((1,), (1,)), ((), ())),
                                preferred_element_type=jnp.float32) * scale
            p = jax.nn.softmax(s, axis=-1)
            parts.append(jnp.dot(p, vh, preferred_element_type=jnp.float32))
        att = jnp.concatenate(parts, axis=1)
        ao = jnp.dot(att, wao_ref[...], preferred_element_type=jnp.float32) + bao_ref[...]
        x = x0 + ao
        h2 = _ln(x, g2_ref[...], b2n_ref[...])
        ffn = jnp.dot(_gelu(jnp.dot(h2, w1_ref[...], preferred_element_type=jnp.float32)
                            + bf1_ref[...]),
                      w2_ref[...], preferred_element_type=jnp.float32)
        x = x + ffn + bf2_ref[...]
        keys = jnp.dot(x, wkr_ref[...], preferred_element_type=jnp.float32) + bkr_ref[...]
        s = lax.dot_general(rq_ref[...], keys, (((1,), (1,)), ((), ())),
                            preferred_element_type=jnp.float32)
        w = jax.nn.softmax(s, axis=-1)
        o_ref[0] = jnp.dot(w, x, preferred_element_type=jnp.float32)

    full = lambda shape: pl.BlockSpec(shape, lambda i: tuple(0 for _ in shape))
    return pl.pallas_call(
        body,
        grid=(NM,),
        in_specs=[
            pl.BlockSpec((1, MA, H), lambda i: (i, 0, 0)),
            full((H, H)), full((1, H)),
            full((H, H)), full((1, H)),
            full((H, H)), full((1, H)),
            full((H, H)), full((1, H)),
            full((1, H)), full((1, H)),
            full((1, H)), full((1, H)),
            full((H, FF)), full((1, FF)),
            full((FF, H)), full((1, H)),
            full((1, H)), full((H, H)), full((1, H)),
        ],
        out_specs=pl.BlockSpec((1, 1, H), lambda i: (i, 0, 0)),
        out_shape=jax.ShapeDtypeStruct((NM, 1, H), jnp.float32),
    )(x3, wqt, bq, wkt, bk, wvt, bv, waot, bao,
      ln1g, ln1b, ln2g, ln2b, w1t, b1, w2t, b2, rq, wkrt, bkr)


def kernel(f_atoms, f_bonds, a2b, b2a, b2revb, a_scope, params):
    p = params
    r2 = lambda a: a.reshape(1, -1)

    wia = p['W_i'][:, :AF].T
    wib = p['W_i'][:, AF:].T
    woa = p['W_o'][:, :AF].T
    wob = p['W_o'][:, AF:].T

    a2b_flat = jnp.concatenate(
        [a2b, jnp.zeros((NAP - NA, NEI), a2b.dtype)], axis=0
    ).reshape(-1)

    ga = _sc_gather_rows(f_atoms, b2a)
    msg = _tc_init(ga, f_bonds, wia, wib, r2(p['b_i']))

    for t in range(DEPTH - 1):
        nei = _sc_gather_sum(msg, a2b_flat)
        nm = _sc_gather_sub(nei, msg, b2a, b2revb)
        msg = _tc_depth(nm, msg, p['W_h'][t].T, r2(p['b_h'][t]),
                        r2(p['msg_g'][t]), r2(p['msg_b'][t]))

    a_msg = _sc_gather_sum(msg, a2b_flat)[:NA]
    ah = _tc_atom(f_atoms, a_msg, woa, wob, r2(p['b_o']),
                  r2(p['an_g']), r2(p['an_b']))

    x3 = ah.reshape(NM, MA, H)
    out = _tc_attn(
        x3, p['Wq'].T, r2(p['bq']), p['Wk'].T, r2(p['bk']),
        p['Wv'].T, r2(p['bv']), p['Wao'].T, r2(p['bao']),
        r2(p['ln1_g']), r2(p['ln1_b']), r2(p['ln2_g']), r2(p['ln2_b']),
        p['W1'].T, r2(p['b1']), p['W2'].T, r2(p['b2']),
        p['rq'].reshape(1, H), p['Wkr'].T, r2(p['bkr']))
    return out.reshape(NM, H)


# speedup vs baseline: 1.6673x; 1.0353x over previous
import functools
import math

import jax
import jax.numpy as jnp
from jax import lax
from jax.experimental import pallas as pl
from jax.experimental.pallas import tpu as pltpu
from jax.experimental.pallas import tpu_sc as plsc

H = 128
AF = 128
BF = 16
NA = 10000
NB = 160000
NEI = 16
NM = 100
MA = 100
DEPTH = 4
NH = 4
DH = H // NH
FF = 2 * H

NC = 2
NS = 16
NW = NC * NS

NAP = 10240
CA = 8
APW = NAP // NW
NCHA = APW // CA

BPW = NB // NW
CB = 128
NCHB = BPW // CB
TB = BPW - NCHB * CB

def _sc_mesh():
    return plsc.VectorSubcoreMesh(core_axis_name="c", subcore_axis_name="s")


def _wid():
    return lax.axis_index("s") * NC + lax.axis_index("c")


def _sc_gather_rows(table, idx):
    n = idx.shape[0]
    per_w = n // NW
    nch = per_w // CB
    npairs = (nch + 1) // 2

    @functools.partial(
        pl.kernel,
        mesh=_sc_mesh(),
        out_type=jax.ShapeDtypeStruct((n, H), jnp.float32),
        scratch_types=[
            pltpu.VMEM((per_w,), jnp.int32),
            pltpu.VMEM((CB, H), jnp.float32),
            pltpu.VMEM((CB, H), jnp.float32),
            pltpu.SemaphoreType.DMA,
            pltpu.SemaphoreType.DMA,
            pltpu.SemaphoreType.DMA,
            pltpu.SemaphoreType.DMA,
        ],
    )
    def k(table_hbm, idx_hbm, out_hbm, i_v, r0, r1, sg0, sg1, ss0, ss1):
        w = _wid()
        base_w = w * per_w
        pltpu.sync_copy(idx_hbm.at[pl.ds(base_w, per_w)], i_v)
        bufs = ((r0, sg0, ss0), (r1, sg1, ss1))

        def issue(ci, b):
            r, sg, _ = bufs[b]
            pltpu.async_copy(table_hbm.at[i_v.at[pl.ds(ci * CB, CB)]], r, sg)

        issue(0, 0)
        issue(1, 1)

        def pair(pi, carry):
            for b in range(2):
                r, sg, ss = bufs[b]
                ci = pi * 2 + b

                @pl.when(ci < nch)
                def _():
                    pltpu.make_async_copy(table_hbm.at[pl.ds(0, CB)], r, sg).wait()
                    pltpu.async_copy(r, out_hbm.at[pl.ds(base_w + ci * CB, CB)], ss)
                    pltpu.make_async_copy(r, out_hbm.at[pl.ds(0, CB)], ss).wait()

                    @pl.when(ci + 2 < nch)
                    def _():
                        issue(ci + 2, b)
            return carry

        lax.fori_loop(0, npairs, pair, 0)

        if TB:
            tb = nch * CB
            pltpu.async_copy(
                table_hbm.at[i_v.at[pl.ds(tb, TB)]], r0.at[pl.ds(0, TB)], sg0
            ).wait()
            pltpu.sync_copy(r0.at[pl.ds(0, TB)],
                            out_hbm.at[pl.ds(base_w + tb, TB)])

    return k(table, idx)


def _sc_gather_sum(msg, a2b_flat):
    @functools.partial(
        pl.kernel,
        mesh=_sc_mesh(),
        out_type=jax.ShapeDtypeStruct((NAP, H), jnp.float32),
        scratch_types=(
            [pltpu.VMEM((APW * NEI,), jnp.int32)]
            + [pltpu.VMEM((CA * NEI, H), jnp.float32) for _ in range(4)]
            + [pltpu.VMEM((CA, H), jnp.float32) for _ in range(4)]
            + [pltpu.SemaphoreType.DMA for _ in range(8)]
        ),
    )
    def k(msg_hbm, a2b_hbm, out_hbm, i_v,
          r0, r1, r2, r3, o0, o1, o2, o3,
          sg0, sg1, sg2, sg3, ss0, ss1, ss2, ss3):
        w = _wid()
        abase_w = w * APW
        pltpu.sync_copy(a2b_hbm.at[pl.ds(abase_w * NEI, APW * NEI)], i_v)
        bufs = ((r0, o0, sg0, ss0), (r1, o1, sg1, ss1),
                (r2, o2, sg2, ss2), (r3, o3, sg3, ss3))

        def issue(ci, b):
            r, _, sg, _ = bufs[b]
            pltpu.async_copy(
                msg_hbm.at[i_v.at[pl.ds(ci * CA * NEI, CA * NEI)]], r, sg)

        for b in range(4):
            issue(b, b)
        nquads = (NCHA + 3) // 4

        def quad(pi, carry):
            for b in range(4):
                r, o, sg, ss = bufs[b]
                ci = pi * 4 + b

                @pl.when(ci < NCHA)
                def _():
                    pltpu.make_async_copy(
                        msg_hbm.at[pl.ds(0, CA * NEI)], r, sg).wait()

                    @pl.when(ci >= 4)
                    def _():
                        pltpu.make_async_copy(
                            o, out_hbm.at[pl.ds(0, CA)], ss).wait()

                    def atom(a, c2):
                        def nb(j, accs):
                            return tuple(
                                accs[kk] + r[a * NEI + j, pl.ds(kk * 16, 16)]
                                for kk in range(8)
                            )

                        accs = lax.fori_loop(
                            0, NEI, nb,
                            tuple(jnp.zeros((16,), jnp.float32)
                                  for _ in range(8)),
                        )
                        for kk in range(8):
                            o[a, pl.ds(kk * 16, 16)] = accs[kk]
                        return c2

                    lax.fori_loop(0, CA, atom, 0)
                    pltpu.async_copy(
                        o, out_hbm.at[pl.ds(abase_w + ci * CA, CA)], ss)

                    @pl.when(ci + 4 < NCHA)
                    def _():
                        issue(ci + 4, b)
            return carry

        lax.fori_loop(0, nquads, quad, 0)
        for b in range(4):
            _, o, _, ss = bufs[b]
            pltpu.make_async_copy(o, out_hbm.at[pl.ds(0, CA)], ss).wait()

    return k(msg, a2b_flat)


def _sc_gather_sub(nei, msg, b2a, b2revb):
    @functools.partial(
        pl.kernel,
        mesh=_sc_mesh(),
        out_type=jax.ShapeDtypeStruct((NB, H), jnp.float32),
        scratch_types=[
            pltpu.VMEM((BPW,), jnp.int32),
            pltpu.VMEM((BPW,), jnp.int32),
            pltpu.VMEM((CB, H), jnp.float32),
            pltpu.VMEM((CB, H), jnp.float32),
            pltpu.VMEM((CB, H), jnp.float32),
            pltpu.VMEM((CB, H), jnp.float32),
            pltpu.VMEM((CB, H), jnp.float32),
            pltpu.VMEM((CB, H), jnp.float32),
            pltpu.SemaphoreType.DMA,
            pltpu.SemaphoreType.DMA,
            pltpu.SemaphoreType.DMA,
            pltpu.SemaphoreType.DMA,
            pltpu.SemaphoreType.DMA,
            pltpu.SemaphoreType.DMA,
        ],
    )
    def k(nei_hbm, msg_hbm, b2a_hbm, b2revb_hbm, out_hbm,
          ia_v, ib_v, ra0, ra1, rb0, rb1, o0, o1,
          sa0, sa1, sb0, sb1, ss0, ss1):
        w = _wid()
        base_w = w * BPW
        pltpu.sync_copy(b2a_hbm.at[pl.ds(base_w, BPW)], ia_v)
        pltpu.sync_copy(b2revb_hbm.at[pl.ds(base_w, BPW)], ib_v)
        bufs = ((ra0, rb0, o0, sa0, sb0, ss0), (ra1, rb1, o1, sa1, sb1, ss1))

        def issue(ci, b):
            ra, rb, _, sa, sb, _ = bufs[b]
            pltpu.async_copy(nei_hbm.at[ia_v.at[pl.ds(ci * CB, CB)]], ra, sa)
            pltpu.async_copy(msg_hbm.at[ib_v.at[pl.ds(ci * CB, CB)]], rb, sb)

        issue(0, 0)
        issue(1, 1)
        npairs = (NCHB + 1) // 2

        def pair(pi, carry):
            for b in range(2):
                ra, rb, o, sa, sb, ss = bufs[b]
                ci = pi * 2 + b

                @pl.when(ci < NCHB)
                def _():
                    pltpu.make_async_copy(
                        nei_hbm.at[pl.ds(0, CB)], ra, sa).wait()
                    pltpu.make_async_copy(
                        msg_hbm.at[pl.ds(0, CB)], rb, sb).wait()

                    @pl.when(ci >= 2)
                    def _():
                        pltpu.make_async_copy(
                            o, out_hbm.at[pl.ds(0, CB)], ss).wait()

                    def row(r_i, c2):
                        for kk in range(8):
                            sl = pl.ds(kk * 16, 16)
                            o[r_i, sl] = ra[r_i, sl] - rb[r_i, sl]
                        return c2

                    lax.fori_loop(0, CB, row, 0)
                    pltpu.async_copy(
                        o, out_hbm.at[pl.ds(base_w + ci * CB, CB)], ss)

                    @pl.when(ci + 2 < NCHB)
                    def _():
                        issue(ci + 2, b)
            return carry

        lax.fori_loop(0, npairs, pair, 0)
        for b in range(2):
            _, _, o, _, _, ss = bufs[b]
            pltpu.make_async_copy(o, out_hbm.at[pl.ds(0, CB)], ss).wait()

        if TB:
            tb = NCHB * CB
            cpa = pltpu.async_copy(
                nei_hbm.at[ia_v.at[pl.ds(tb, TB)]], ra0.at[pl.ds(0, TB)], sa0)
            cpb = pltpu.async_copy(
                msg_hbm.at[ib_v.at[pl.ds(tb, TB)]], rb0.at[pl.ds(0, TB)], sb0)
            cpa.wait()
            cpb.wait()

            def trow(r_i, c2):
                for kk in range(8):
                    sl = pl.ds(kk * 16, 16)
                    o0[r_i, sl] = ra0[r_i, sl] - rb0[r_i, sl]
                return c2

            lax.fori_loop(0, TB, trow, 0)
            pltpu.sync_copy(o0.at[pl.ds(0, TB)],
                            out_hbm.at[pl.ds(base_w + tb, TB)])

    return k(nei, msg, b2a, b2revb)


def _ln(x, g, b):
    m = jnp.mean(x, -1, keepdims=True)
    v = jnp.mean((x - m) ** 2, -1, keepdims=True)
    return (x - m) * lax.rsqrt(v + 1e-5) * g + b


def _gelu(x):
    return 0.5 * x * (1.0 + lax.erf(x * (1.0 / math.sqrt(2.0))))


BLK = 2000
BLKA = 2000


def _tc_init(ga, f_bonds, wia, wib, bi):
    def body(ga_ref, fb_ref, wa_ref, wb_ref, bi_ref, o_ref):
        x = (jnp.dot(ga_ref[...], wa_ref[...], preferred_element_type=jnp.float32)
             + jnp.dot(fb_ref[...], wb_ref[...], preferred_element_type=jnp.float32)
             + bi_ref[...])
        o_ref[...] = _gelu(x)

    return pl.pallas_call(
        body,
        grid=(NB // BLK,),
        in_specs=[
            pl.BlockSpec((BLK, AF), lambda i: (i, 0)),
            pl.BlockSpec((BLK, BF), lambda i: (i, 0)),
            pl.BlockSpec((AF, H), lambda i: (0, 0)),
            pl.BlockSpec((BF, H), lambda i: (0, 0)),
            pl.BlockSpec((1, H), lambda i: (0, 0)),
        ],
        out_specs=pl.BlockSpec((BLK, H), lambda i: (i, 0)),
        out_shape=jax.ShapeDtypeStruct((NB, H), jnp.float32),
    )(ga, f_bonds, wia, wib, bi)


def _tc_depth(nm, msg, wh, bh, g, b):
    def body(nm_ref, msg_ref, wh_ref, bh_ref, g_ref, b_ref, o_ref):
        xn = _ln(nm_ref[...], g_ref[...], b_ref[...])
        y = jnp.dot(xn, wh_ref[...], preferred_element_type=jnp.float32) + bh_ref[...]
        o_ref[...] = msg_ref[...] + _gelu(y)

    return pl.pallas_call(
        body,
        grid=(NB // BLK,),
        in_specs=[
            pl.BlockSpec((BLK, H), lambda i: (i, 0)),
            pl.BlockSpec((BLK, H), lambda i: (i, 0)),
            pl.BlockSpec((H, H), lambda i: (0, 0)),
            pl.BlockSpec((1, H), lambda i: (0, 0)),
            pl.BlockSpec((1, H), lambda i: (0, 0)),
            pl.BlockSpec((1, H), lambda i: (0, 0)),
        ],
        out_specs=pl.BlockSpec((BLK, H), lambda i: (i, 0)),
        out_shape=jax.ShapeDtypeStruct((NB, H), jnp.float32),
    )(nm, msg, wh, bh, g, b)


def _tc_atom(f_atoms, a_msg, woa, wob, bo, ang, anb):
    def body(fa_ref, am_ref, wa_ref, wb_ref, bo_ref, g_ref, b_ref, o_ref):
        x = (jnp.dot(fa_ref[...], wa_ref[...], preferred_element_type=jnp.float32)
             + jnp.dot(am_ref[...], wb_ref[...], preferred_element_type=jnp.float32)
             + bo_ref[...])
        o_ref[...] = _ln(_gelu(x), g_ref[...], b_ref[...])

    return pl.pallas_call(
        body,
        grid=(NA // BLKA,),
        in_specs=[
            pl.BlockSpec((BLKA, AF), lambda i: (i, 0)),
            pl.BlockSpec((BLKA, H), lambda i: (i, 0)),
            pl.BlockSpec((AF, H), lambda i: (0, 0)),
            pl.BlockSpec((H, H), lambda i: (0, 0)),
            pl.BlockSpec((1, H), lambda i: (0, 0)),
            pl.BlockSpec((1, H), lambda i: (0, 0)),
            pl.BlockSpec((1, H), lambda i: (0, 0)),
        ],
        out_specs=pl.BlockSpec((BLKA, H), lambda i: (i, 0)),
        out_shape=jax.ShapeDtypeStruct((NA, H), jnp.float32),
    )(f_atoms, a_msg, woa, wob, bo, ang, anb)


def _tc_attn(x3, wqt, bq, wkt, bk, wvt, bv, waot, bao,
             ln1g, ln1b, ln2g, ln2b, w1t, b1, w2t, b2, rq, wkrt, bkr):
    def body(x_ref, wq_ref, bq_ref, wk_ref, bk_ref, wv_ref, bv_ref,
             wao_ref, bao_ref, g1_ref, b1n_ref, g2_ref, b2n_ref,
             w1_ref, bf1_ref, w2_ref, bf2_ref, rq_ref, wkr_ref, bkr_ref,
             o_ref):
        x0 = x_ref[0]
        h = _ln(x0, g1_ref[...], b1n_ref[...])
        q = jnp.dot(h, wq_ref[...], preferred_element_type=jnp.float32) + bq_ref[...]
        kk = jnp.dot(h, wk_ref[...], preferred_element_type=jnp.float32) + bk_ref[...]
        v = jnp.dot(h, wv_ref[...], preferred_element_type=jnp.float32) + bv_ref[...]
        scale = 1.0 / math.sqrt(DH)
        parts = []
        for hd in range(NH):
            sl = slice(hd * DH, (hd + 1) * DH)
            qh = q[:, sl]
            khd = kk[:, sl]
            vh = v[:, sl]
            s = lax.dot_general(qh, khd, (((1,), (1,)), ((), ())),
                                preferred_element_type=jnp.float32) * scale
            p = jax.nn.softmax(s, axis=-1)
            parts.append(jnp.dot(p, vh, preferred_element_type=jnp.float32))
        att = jnp.concatenate(parts, axis=1)
        ao = jnp.dot(att, wao_ref[...], preferred_element_type=jnp.float32) + bao_ref[...]
        x = x0 + ao
        h2 = _ln(x, g2_ref[...], b2n_ref[...])
        ffn = jnp.dot(_gelu(jnp.dot(h2, w1_ref[...], preferred_element_type=jnp.float32)
                            + bf1_ref[...]),
                      w2_ref[...], preferred_element_type=jnp.float32)
        x = x + ffn + bf2_ref[...]
        keys = jnp.dot(x, wkr_ref[...], preferred_element_type=jnp.float32) + bkr_ref[...]
        s = lax.dot_general(rq_ref[...], keys, (((1,), (1,)), ((), ())),
                            preferred_element_type=jnp.float32)
        w = jax.nn.softmax(s, axis=-1)
        o_ref[0] = jnp.dot(w, x, preferred_element_type=jnp.float32)

    full = lambda shape: pl.BlockSpec(shape, lambda i: tuple(0 for _ in shape))
    return pl.pallas_call(
        body,
        grid=(NM,),
        in_specs=[
            pl.BlockSpec((1, MA, H), lambda i: (i, 0, 0)),
            full((H, H)), full((1, H)),
            full((H, H)), full((1, H)),
            full((H, H)), full((1, H)),
            full((H, H)), full((1, H)),
            full((1, H)), full((1, H)),
            full((1, H)), full((1, H)),
            full((H, FF)), full((1, FF)),
            full((FF, H)), full((1, H)),
            full((1, H)), full((H, H)), full((1, H)),
        ],
        out_specs=pl.BlockSpec((1, 1, H), lambda i: (i, 0, 0)),
        out_shape=jax.ShapeDtypeStruct((NM, 1, H), jnp.float32),
    )(x3, wqt, bq, wkt, bk, wvt, bv, waot, bao,
      ln1g, ln1b, ln2g, ln2b, w1t, b1, w2t, b2, rq, wkrt, bkr)


def kernel(f_atoms, f_bonds, a2b, b2a, b2revb, a_scope, params):
    p = params
    r2 = lambda a: a.reshape(1, -1)

    wia = p['W_i'][:, :AF].T
    wib = p['W_i'][:, AF:].T
    woa = p['W_o'][:, :AF].T
    wob = p['W_o'][:, AF:].T

    a2b_flat = jnp.concatenate(
        [a2b, jnp.zeros((NAP - NA, NEI), a2b.dtype)], axis=0
    ).reshape(-1)

    ga = _sc_gather_rows(f_atoms, b2a)
    msg = _tc_init(ga, f_bonds, wia, wib, r2(p['b_i']))

    for t in range(DEPTH - 1):
        nei = _sc_gather_sum(msg, a2b_flat)
        nm = _sc_gather_sub(nei, msg, b2a, b2revb)
        msg = _tc_depth(nm, msg, p['W_h'][t].T, r2(p['b_h'][t]),
                        r2(p['msg_g'][t]), r2(p['msg_b'][t]))

    a_msg = _sc_gather_sum(msg, a2b_flat)[:NA]
    ah = _tc_atom(f_atoms, a_msg, woa, wob, r2(p['b_o']),
                  r2(p['an_g']), r2(p['an_b']))

    x3 = ah.reshape(NM, MA, H)
    out = _tc_attn(
        x3, p['Wq'].T, r2(p['bq']), p['Wk'].T, r2(p['bk']),
        p['Wv'].T, r2(p['bv']), p['Wao'].T, r2(p['bao']),
        r2(p['ln1_g']), r2(p['ln1_b']), r2(p['ln2_g']), r2(p['ln2_b']),
        p['W1'].T, r2(p['b1']), p['W2'].T, r2(p['b2']),
        p['rq'].reshape(1, H), p['Wkr'].T, r2(p['bkr']))
    return out.reshape(NM, H)
